# baseline (speedup 1.0000x reference)
import functools

import jax
import jax.numpy as jnp
from jax import lax
from jax.experimental import pallas as pl
from jax.experimental.pallas import tpu as pltpu
from jax.experimental.pallas import tpu_sc as plsc

B = 4
L = 4096
D = 1024

_info = plsc.get_sparse_core_info()
_NC = _info.num_cores
_NS = _info.num_subcores
_NW = _NC * _NS
_ROWS = L // _NW
_CHUNK = 32
_NCH = _ROWS // _CHUNK

_mesh = plsc.VectorSubcoreMesh(core_axis_name="c", subcore_axis_name="s")


@functools.partial(
    pl.kernel,
    out_type=jax.ShapeDtypeStruct((B * L, D), jnp.float32),
    mesh=_mesh,
    scratch_types=[
        pltpu.VMEM((_CHUNK, D), jnp.float32),
        pltpu.VMEM((_CHUNK, D), jnp.float32),
        pltpu.SemaphoreType.DMA,
        pltpu.SemaphoreType.DMA,
        pltpu.SemaphoreType.DMA,
        pltpu.SemaphoreType.DMA,
    ],
)
def _bcast_copy(table_hbm, out_hbm, buf0, buf1, rsem0, rsem1, wsem0, wsem1):
    wid = lax.axis_index("s") * _NC + lax.axis_index("c")
    base = wid * _ROWS
    bufs = (buf0, buf1)
    rsems = (rsem0, rsem1)
    wsems = (wsem0, wsem1)
    reads = [None, None]
    writes = [[], []]
    reads[0] = pltpu.async_copy(table_hbm.at[pl.ds(base, _CHUNK)], buf0, rsem0)
    for c in range(_NCH):
        i = c % 2
        reads[i].wait()
        if c + 1 < _NCH:
            j = (c + 1) % 2
            for w in writes[j]:
                w.wait()
            writes[j] = []
            reads[j] = pltpu.async_copy(
                table_hbm.at[pl.ds(base + (c + 1) * _CHUNK, _CHUNK)],
                bufs[j], rsems[j])
        off = base + c * _CHUNK
        for b in range(B):
            writes[i].append(pltpu.async_copy(
                bufs[i], out_hbm.at[pl.ds(b * L + off, _CHUNK)], wsems[i]))
    for ws in writes:
        for w in ws:
            w.wait()


def kernel(x, table):
    del x
    out = _bcast_copy(table)
    return out.reshape(B, L, D)
